# baseline (speedup 1.0000x reference)
import functools

import jax
import jax.numpy as jnp
from jax import lax
from jax.experimental import pallas as pl
from jax.experimental.pallas import tpu as pltpu
from jax.experimental.pallas import tpu_sc as plsc

N_CORES = 2
N_SUB = 16
NW = N_CORES * N_SUB

CH = 80


def _proj_body(x_ref, wl_ref, wr_ref, xl_ref, xr_ref):
    xb = x_ref[...]
    xl_ref[...] = jnp.dot(xb, wl_ref[...], preferred_element_type=jnp.float32)
    xr_ref[...] = jnp.dot(xb, wr_ref[...], preferred_element_type=jnp.float32)


def _tc_proj(x, wl, wr, blk):
    n, d = x.shape
    h = wl.shape[1]
    grid = n // blk
    return pl.pallas_call(
        _proj_body,
        grid=(grid,),
        in_specs=[
            pl.BlockSpec((blk, d), lambda i: (i, 0)),
            pl.BlockSpec((d, h), lambda i: (0, 0)),
            pl.BlockSpec((d, h), lambda i: (0, 0)),
        ],
        out_specs=[
            pl.BlockSpec((blk, h), lambda i: (i, 0)),
            pl.BlockSpec((blk, h), lambda i: (i, 0)),
        ],
        out_shape=[
            jax.ShapeDtypeStruct((n, h), jnp.float32),
            jax.ShapeDtypeStruct((n, h), jnp.float32),
        ],
    )(x, wl, wr)


def _combine_body(pr_ref, pd_ref, b_ref, wl_ref, wr_ref, xl_ref, xr_ref):
    num = pr_ref[0] + pr_ref[1]
    den = jnp.sum(pd_ref[...], axis=(0, 2))[:, None]
    h = num / (den + 1e-16) + b_ref[...]
    h = jnp.maximum(h, 0.0)
    xl_ref[...] = jnp.dot(h, wl_ref[...], preferred_element_type=jnp.float32)
    xr_ref[...] = jnp.dot(h, wr_ref[...], preferred_element_type=jnp.float32)


def _tc_combine_proj(pr, pd, b, wl, wr, blk):
    n = pr.shape[1]
    h = wl.shape[1]
    grid = n // blk
    return pl.pallas_call(
        _combine_body,
        grid=(grid,),
        in_specs=[
            pl.BlockSpec((2, blk, 128), lambda i: (0, i, 0)),
            pl.BlockSpec((2, blk, 16), lambda i: (0, i, 0)),
            pl.BlockSpec((1, 128), lambda i: (0, 0)),
            pl.BlockSpec((128, h), lambda i: (0, 0)),
            pl.BlockSpec((128, h), lambda i: (0, 0)),
        ],
        out_specs=[
            pl.BlockSpec((blk, h), lambda i: (i, 0)),
            pl.BlockSpec((blk, h), lambda i: (i, 0)),
        ],
        out_shape=[
            jax.ShapeDtypeStruct((n, h), jnp.float32),
            jax.ShapeDtypeStruct((n, h), jnp.float32),
        ],
    )(pr, pd, b.reshape(1, 128), wl, wr)


def _edge_emb_body(ea_ref, w1_ref, w2_ref, e1_ref, e2_ref):
    ea = ea_ref[...]
    e1_ref[...] = jnp.dot(ea, w1_ref[...], preferred_element_type=jnp.float32)
    e2_ref[...] = jnp.dot(ea, w2_ref[...], preferred_element_type=jnp.float32)


def _tc_edge_emb(edge_attr, we1, we2, blk):
    e, ed = edge_attr.shape
    h = we1.shape[1]
    grid = e // blk
    return pl.pallas_call(
        _edge_emb_body,
        grid=(grid,),
        in_specs=[
            pl.BlockSpec((blk, ed), lambda i: (i, 0)),
            pl.BlockSpec((ed, h), lambda i: (0, 0)),
            pl.BlockSpec((ed, h), lambda i: (0, 0)),
        ],
        out_specs=[
            pl.BlockSpec((blk, h), lambda i: (i, 0)),
            pl.BlockSpec((blk, h), lambda i: (i, 0)),
        ],
        out_shape=[
            jax.ShapeDtypeStruct((e, h), jnp.float32),
            jax.ShapeDtypeStruct((e, h), jnp.float32),
        ],
    )(edge_attr, we1, we2)


def _final_body(pr_ref, pd_ref, b_ref, batch_ref, wfc_ref, bfc_ref, out_ref,
                sums_ref, cnts_ref, nb):
    i = pl.program_id(0)

    @pl.when(i == 0)
    def _():
        sums_ref[...] = jnp.zeros_like(sums_ref)
        cnts_ref[...] = jnp.zeros_like(cnts_ref)

    num = pr_ref[0] + pr_ref[1]
    den = jnp.sum(pd_ref[...], axis=(0, 2))[:, None]
    h = num / (den + 1e-16) + b_ref[...]
    h = jnp.maximum(h, 0.0)

    bblk = batch_ref[0, 0, :]
    blk = h.shape[0]
    seg = lax.broadcasted_iota(jnp.int32, (nb, blk), 0)
    oh = (seg == bblk[None, :]).astype(jnp.float32)
    sums_ref[...] += jnp.dot(oh, h, preferred_element_type=jnp.float32)
    cnts_ref[...] += jnp.sum(oh, axis=1, keepdims=True)

    @pl.when(i == pl.num_programs(0) - 1)
    def _():
        mean = sums_ref[...] / jnp.maximum(cnts_ref[...], 1.0)
        out_ref[...] = (
            jnp.dot(mean, wfc_ref[...], preferred_element_type=jnp.float32)
            + bfc_ref[...]
        )


def _tc_final(pr, pd, b, batch, wfc, bfc, nb, blk):
    n = pr.shape[1]
    o = wfc.shape[1]
    grid = n // blk
    return pl.pallas_call(
        functools.partial(_final_body, nb=nb),
        grid=(grid,),
        in_specs=[
            pl.BlockSpec((2, blk, 128), lambda i: (0, i, 0)),
            pl.BlockSpec((2, blk, 16), lambda i: (0, i, 0)),
            pl.BlockSpec((1, 128), lambda i: (0, 0)),
            pl.BlockSpec((1, 1, blk), lambda i: (i, 0, 0)),
            pl.BlockSpec((128, o), lambda i: (0, 0)),
            pl.BlockSpec((1, o), lambda i: (0, 0)),
        ],
        out_specs=pl.BlockSpec((nb, o), lambda i: (0, 0)),
        out_shape=jax.ShapeDtypeStruct((nb, o), jnp.float32),
        scratch_shapes=[
            pltpu.VMEM((nb, 128), jnp.float32),
            pltpu.VMEM((nb, 1), jnp.float32),
        ],
    )(pr, pd, b.reshape(1, 128), batch.reshape(n // blk, 1, blk), wfc,
      bfc.reshape(1, o))


def _sc_edge_rows(src, dst, xl, xr, ee, att, lanes):
    n = xl.shape[0]
    e = ee.shape[0]
    CHA = 40
    epw = e // NW
    nch = epw // CHA
    zch = n // CHA

    mesh = plsc.VectorSubcoreMesh(core_axis_name="c", subcore_axis_name="s")

    BCH = 50
    nblk = nch // BCH

    buf = lambda: [
        pltpu.VMEM((CHA,), jnp.int32),
        pltpu.VMEM((CHA, 128), jnp.float32),
        pltpu.VMEM((CHA, 128), jnp.float32),
        pltpu.VMEM((CHA, 128), jnp.float32),
        pltpu.SemaphoreType.DMA,
        pltpu.SemaphoreType.DMA,
        pltpu.SemaphoreType.DMA,
        pltpu.SemaphoreType.DMA,
    ]

    @functools.partial(
        pl.kernel,
        out_type=(
            jax.ShapeDtypeStruct((N_CORES, n, 128), jnp.float32),
            jax.ShapeDtypeStruct((e,), jnp.float32),
        ),
        mesh=mesh,
        scratch_types=buf() + buf() + [
            pltpu.VMEM((BCH * CHA,), jnp.int32),
            pltpu.VMEM((BCH * CHA,), jnp.int32),
            pltpu.VMEM((BCH * CHA,), jnp.float32),
            pltpu.VMEM((128,), jnp.float32),
            pltpu.VMEM((16,), jnp.int32),
            pltpu.VMEM((192,), jnp.float32),
            pltpu.VMEM_SHARED((n, 128), jnp.float32),
        ],
    )
    def k(src_h, dst_h, xl_h, xr_h, ee_h, att_h, lanes_h, orow_h, ex_h,
          *sc):
        bufA, bufB = sc[0:8], sc[8:16]
        src_blk, dst_blk, e_blk, att_v, lanes_v, s_v, acc_r = sc[16:23]
        c = lax.axis_index("c")
        s = lax.axis_index("s")
        wid = c * N_SUB + s

        pltpu.sync_copy(att_h, att_v)
        pltpu.sync_copy(lanes_h, lanes_v)
        att_regs = [att_v[pl.ds(r * 16, 16)] for r in range(8)]
        lanes = lanes_v[...]
        zero16 = jnp.zeros((16,), jnp.float32)
        for w in range(12):
            s_v[pl.ds(w * 16, 16)] = zero16

        z_v = bufA[1]
        zB_v = bufB[1]

        @pl.loop(0, CHA)
        def _(i):
            for r in range(8):
                z_v[i, pl.ds(r * 16, 16)] = zero16
                zB_v[i, pl.ds(r * 16, 16)] = zero16

        zero16i = jnp.zeros((16,), jnp.int32)
        bufB[0][pl.ds(0, 16)] = zero16i
        bufB[0][pl.ds(16, 16)] = zero16i
        bufB[0][pl.ds(24, 16)] = zero16i

        @pl.loop(s, zch, step=N_SUB)
        def _(t):
            pltpu.sync_copy(z_v, acc_r.at[pl.ds(t * CHA, CHA), :])

        plsc.subcore_barrier()

        pltpu.async_copy(zB_v, acc_r.at[bufB[0]], bufB[7], add=True)

        ebase0 = wid * epw

        def issue(rc, f, b):
            base = ebase0 + (f * BCH + 0) * CHA + rc * CHA
            siv = src_blk.at[pl.ds(rc * CHA, CHA)]
            div = dst_blk.at[pl.ds(rc * CHA, CHA)]
            pltpu.async_copy(xl_h.at[siv], b[1], b[5])
            pltpu.async_copy(xr_h.at[div], b[2], b[6])
            pltpu.async_copy(ee_h.at[pl.ds(base, CHA), :], b[3], b[7])

        def wait_gathers(b):
            siv = src_blk.at[pl.ds(0, CHA)]
            div = dst_blk.at[pl.ds(0, CHA)]
            pltpu.make_async_copy(xl_h.at[siv], b[1], b[5]).wait()
            pltpu.make_async_copy(xr_h.at[div], b[2], b[6]).wait()
            pltpu.make_async_copy(ee_h.at[pl.ds(0, CHA), :], b[3], b[7]).wait()

        def compute(rc, f, b):
            dstS, a_v, b_v, c_v = b[0:4]
            wait_gathers(b)
            off = rc * CHA
            dstS[pl.ds(0, 16)] = dst_blk[pl.ds(off, 16)]
            dstS[pl.ds(16, 16)] = dst_blk[pl.ds(off + 16, 16)]
            dstS[pl.ds(24, 16)] = dst_blk[pl.ds(off + 24, 16)]

            @pl.loop(0, CHA // 16)
            def _(g):
                def quad(qd, exg):
                    for u in range(4):
                        j = qd * 4 + u
                        i = g * 16 + j
                        w0 = 48 * u
                        acc = zero16
                        arows = []
                        for r in range(8):
                            av = a_v[i, pl.ds(r * 16, 16)]
                            m = (av + b_v[i, pl.ds(r * 16, 16)]
                                 + c_v[i, pl.ds(r * 16, 16)])
                            m = jnp.maximum(m, m * 0.2)
                            acc = acc + m * att_regs[r]
                            arows.append(av)
                        for kk in (8, 4, 2, 1):
                            s_v[pl.ds(w0 + 16, 16)] = acc
                            acc = (acc + s_v[pl.ds(w0 + 16 + kk, 16)]
                                   + s_v[pl.ds(w0 + kk, 16)])
                        ex = jnp.exp(acc)
                        for r in range(8):
                            a_v[i, pl.ds(r * 16, 16)] = arows[r] * ex
                        exg = jnp.where(lanes == j, ex, exg)
                    return exg

                exg = lax.fori_loop(0, 4, quad, zero16)
                e_blk[pl.ds(rc * CHA + g * 16, 16)] = exg

            pltpu.async_copy(a_v, acc_r.at[dstS], b[7], add=True)

        def wait_scatter(b):
            pltpu.make_async_copy(b[1], acc_r.at[b[0]], b[7]).wait()

        for f in range(nblk):
            fb = ebase0 + f * BCH * CHA
            pltpu.sync_copy(src_h.at[pl.ds(fb, BCH * CHA)], src_blk)
            pltpu.sync_copy(dst_h.at[pl.ds(fb, BCH * CHA)], dst_blk)
            issue(0, f, bufA)

            @pl.loop(0, BCH, step=2)
            def _(rc):
                wait_scatter(bufB)
                issue(rc + 1, f, bufB)
                compute(rc, f, bufA)
                compute(rc + 1, f, bufB)
                wait_scatter(bufA)
                issue(jnp.minimum(rc + 2, BCH - 1), f, bufA)

            wait_gathers(bufA)
            pltpu.sync_copy(e_blk, ex_h.at[pl.ds(fb, BCH * CHA)])

        wait_scatter(bufB)
        plsc.subcore_barrier()

        @pl.loop(s, zch, step=N_SUB)
        def _(t):
            pltpu.sync_copy(acc_r.at[pl.ds(t * CHA, CHA), :], z_v)
            pltpu.sync_copy(z_v, orow_h.at[c, pl.ds(t * CHA, CHA), :])

    return k(src, dst, xl, xr, ee, att, lanes)


def _sc_edge_denom(dst, ex, n, lanes):
    e = ex.shape[0]
    epw = e // NW
    nch = epw // CH
    zch = n // CH

    mesh = plsc.VectorSubcoreMesh(core_axis_name="c", subcore_axis_name="s")

    dbuf = lambda: [
        pltpu.VMEM((CH,), jnp.int32),
        pltpu.VMEM((CH,), jnp.float32),
        pltpu.VMEM((CH, 16), jnp.float32),
        pltpu.SemaphoreType.DMA,
        pltpu.SemaphoreType.DMA,
    ]

    @functools.partial(
        pl.kernel,
        out_type=jax.ShapeDtypeStruct((N_CORES, n, 16), jnp.float32),
        mesh=mesh,
        scratch_types=dbuf() + dbuf() + [
            pltpu.VMEM((16,), jnp.int32),
            pltpu.VMEM_SHARED((n, 16), jnp.float32),
        ],
    )
    def k(dst_h, ex_h, lanes_h, oden_h, *sc):
        bufA, bufB = sc[0:5], sc[5:10]
        lanes_v, acc_d = sc[10:12]
        c = lax.axis_index("c")
        s = lax.axis_index("s")
        wid = c * N_SUB + s

        pltpu.sync_copy(lanes_h, lanes_v)
        lanes = lanes_v[...]
        zero16 = jnp.zeros((16,), jnp.float32)
        zd_v = bufA[2]

        @pl.loop(0, CH)
        def _(i):
            zd_v[i, :] = zero16

        @pl.loop(s, zch, step=N_SUB)
        def _(t):
            pltpu.sync_copy(zd_v, acc_d.at[pl.ds(t * CH, CH), :])

        plsc.subcore_barrier()

        ebase0 = wid * epw

        def issue(ch, b):
            base = ebase0 + ch * CH
            pltpu.async_copy(dst_h.at[pl.ds(base, CH)], b[0], b[3])
            pltpu.async_copy(ex_h.at[pl.ds(base, CH)], b[1], b[4])

        def compute(ch, b):
            dst_v, e_v, d_v = b[0:3]
            pltpu.make_async_copy(dst_h.at[pl.ds(0, CH)], dst_v, b[3]).wait()
            pltpu.make_async_copy(ex_h.at[pl.ds(0, CH)], e_v, b[4]).wait()

            @pl.loop(0, CH // 16)
            def _(g):
                ex16 = e_v[pl.ds(g * 16, 16)]

                @pl.loop(0, 16)
                def _(j):
                    d_v[g * 16 + j, :] = jnp.where(lanes == j, ex16, 0.0)

            pltpu.sync_copy(d_v, acc_d.at[dst_v], add=True)

        issue(0, bufA)

        @pl.loop(0, nch - 1, step=2)
        def _(ch):
            issue(ch + 1, bufB)
            compute(ch, bufA)
            issue(ch + 2, bufA)
            compute(ch + 1, bufB)

        compute(nch - 1, bufA)

        plsc.subcore_barrier()

        @pl.loop(s, zch, step=N_SUB)
        def _(t):
            pltpu.sync_copy(acc_d.at[pl.ds(t * CH, CH), :], zd_v)
            pltpu.sync_copy(zd_v, oden_h.at[c, pl.ds(t * CH, CH), :])

    return k(dst, ex, lanes)


def kernel(x, edge_index, edge_attr, batch, Wl1, Wr1, We1, att1, b1,
           Wl2, Wr2, We2, att2, b2, Wfc, bfc):
    n = x.shape[0]
    nb = 64
    src = edge_index[0]
    dst = edge_index[1]
    lanes = jnp.arange(16, dtype=jnp.int32)

    xl1, xr1 = _tc_proj(x, Wl1, Wr1, blk=1000)
    e1, e2 = _tc_edge_emb(edge_attr, We1, We2, blk=8000)

    pr1, ex1 = _sc_edge_rows(src, dst, xl1, xr1, e1, att1, lanes)
    pd1 = _sc_edge_denom(dst, ex1, n, lanes)
    xl2, xr2 = _tc_combine_proj(pr1, pd1, b1, Wl2, Wr2, blk=1000)

    pr2, ex2 = _sc_edge_rows(src, dst, xl2, xr2, e2, att2, lanes)
    pd2 = _sc_edge_denom(dst, ex2, n, lanes)
    return _tc_final(pr2, pd2, b2, batch, Wfc, bfc, nb=nb, blk=1000)


# speedup vs baseline: 1.1712x; 1.1712x over previous
import functools

import jax
import jax.numpy as jnp
from jax import lax
from jax.experimental import pallas as pl
from jax.experimental.pallas import tpu as pltpu
from jax.experimental.pallas import tpu_sc as plsc

N_CORES = 2
N_SUB = 16
NW = N_CORES * N_SUB

CH = 80


def _proj_body(x_ref, wl_ref, wr_ref, xl_ref, xr_ref):
    xb = x_ref[...]
    xl_ref[...] = jnp.dot(xb, wl_ref[...], preferred_element_type=jnp.float32)
    xr_ref[...] = jnp.dot(xb, wr_ref[...], preferred_element_type=jnp.float32)


def _tc_proj(x, wl, wr, blk):
    n, d = x.shape
    h = wl.shape[1]
    grid = n // blk
    return pl.pallas_call(
        _proj_body,
        grid=(grid,),
        in_specs=[
            pl.BlockSpec((blk, d), lambda i: (i, 0)),
            pl.BlockSpec((d, h), lambda i: (0, 0)),
            pl.BlockSpec((d, h), lambda i: (0, 0)),
        ],
        out_specs=[
            pl.BlockSpec((blk, h), lambda i: (i, 0)),
            pl.BlockSpec((blk, h), lambda i: (i, 0)),
        ],
        out_shape=[
            jax.ShapeDtypeStruct((n, h), jnp.float32),
            jax.ShapeDtypeStruct((n, h), jnp.float32),
        ],
    )(x, wl, wr)


def _combine_body(pr_ref, pd_ref, b_ref, wl_ref, wr_ref, xl_ref, xr_ref):
    num = pr_ref[0] + pr_ref[1]
    den = jnp.sum(pd_ref[...], axis=(0, 2))[:, None]
    h = num / (den + 1e-16) + b_ref[...]
    h = jnp.maximum(h, 0.0)
    xl_ref[...] = jnp.dot(h, wl_ref[...], preferred_element_type=jnp.float32)
    xr_ref[...] = jnp.dot(h, wr_ref[...], preferred_element_type=jnp.float32)


def _tc_combine_proj(pr, pd, b, wl, wr, blk):
    n = pr.shape[1]
    h = wl.shape[1]
    grid = n // blk
    return pl.pallas_call(
        _combine_body,
        grid=(grid,),
        in_specs=[
            pl.BlockSpec((2, blk, 128), lambda i: (0, i, 0)),
            pl.BlockSpec((2, blk, 16), lambda i: (0, i, 0)),
            pl.BlockSpec((1, 128), lambda i: (0, 0)),
            pl.BlockSpec((128, h), lambda i: (0, 0)),
            pl.BlockSpec((128, h), lambda i: (0, 0)),
        ],
        out_specs=[
            pl.BlockSpec((blk, h), lambda i: (i, 0)),
            pl.BlockSpec((blk, h), lambda i: (i, 0)),
        ],
        out_shape=[
            jax.ShapeDtypeStruct((n, h), jnp.float32),
            jax.ShapeDtypeStruct((n, h), jnp.float32),
        ],
    )(pr, pd, b.reshape(1, 128), wl, wr)


def _edge_emb_body(ea_ref, w1_ref, w2_ref, e1_ref, e2_ref):
    ea = ea_ref[...]
    e1_ref[...] = jnp.dot(ea, w1_ref[...], preferred_element_type=jnp.float32)
    e2_ref[...] = jnp.dot(ea, w2_ref[...], preferred_element_type=jnp.float32)


def _tc_edge_emb(edge_attr, we1, we2, blk):
    e, ed = edge_attr.shape
    h = we1.shape[1]
    grid = e // blk
    return pl.pallas_call(
        _edge_emb_body,
        grid=(grid,),
        in_specs=[
            pl.BlockSpec((blk, ed), lambda i: (i, 0)),
            pl.BlockSpec((ed, h), lambda i: (0, 0)),
            pl.BlockSpec((ed, h), lambda i: (0, 0)),
        ],
        out_specs=[
            pl.BlockSpec((blk, h), lambda i: (i, 0)),
            pl.BlockSpec((blk, h), lambda i: (i, 0)),
        ],
        out_shape=[
            jax.ShapeDtypeStruct((e, h), jnp.float32),
            jax.ShapeDtypeStruct((e, h), jnp.float32),
        ],
    )(edge_attr, we1, we2)


def _final_body(pr_ref, pd_ref, b_ref, batch_ref, wfc_ref, bfc_ref, out_ref,
                sums_ref, cnts_ref, nb):
    i = pl.program_id(0)

    @pl.when(i == 0)
    def _():
        sums_ref[...] = jnp.zeros_like(sums_ref)
        cnts_ref[...] = jnp.zeros_like(cnts_ref)

    num = pr_ref[0] + pr_ref[1]
    den = jnp.sum(pd_ref[...], axis=(0, 2))[:, None]
    h = num / (den + 1e-16) + b_ref[...]
    h = jnp.maximum(h, 0.0)

    bblk = batch_ref[0, 0, :]
    blk = h.shape[0]
    seg = lax.broadcasted_iota(jnp.int32, (nb, blk), 0)
    oh = (seg == bblk[None, :]).astype(jnp.float32)
    sums_ref[...] += jnp.dot(oh, h, preferred_element_type=jnp.float32)
    cnts_ref[...] += jnp.sum(oh, axis=1, keepdims=True)

    @pl.when(i == pl.num_programs(0) - 1)
    def _():
        mean = sums_ref[...] / jnp.maximum(cnts_ref[...], 1.0)
        out_ref[...] = (
            jnp.dot(mean, wfc_ref[...], preferred_element_type=jnp.float32)
            + bfc_ref[...]
        )


def _tc_final(pr, pd, b, batch, wfc, bfc, nb, blk):
    n = pr.shape[1]
    o = wfc.shape[1]
    grid = n // blk
    return pl.pallas_call(
        functools.partial(_final_body, nb=nb),
        grid=(grid,),
        in_specs=[
            pl.BlockSpec((2, blk, 128), lambda i: (0, i, 0)),
            pl.BlockSpec((2, blk, 16), lambda i: (0, i, 0)),
            pl.BlockSpec((1, 128), lambda i: (0, 0)),
            pl.BlockSpec((1, 1, blk), lambda i: (i, 0, 0)),
            pl.BlockSpec((128, o), lambda i: (0, 0)),
            pl.BlockSpec((1, o), lambda i: (0, 0)),
        ],
        out_specs=pl.BlockSpec((nb, o), lambda i: (0, 0)),
        out_shape=jax.ShapeDtypeStruct((nb, o), jnp.float32),
        scratch_shapes=[
            pltpu.VMEM((nb, 128), jnp.float32),
            pltpu.VMEM((nb, 1), jnp.float32),
        ],
    )(pr, pd, b.reshape(1, 128), batch.reshape(n // blk, 1, blk), wfc,
      bfc.reshape(1, o))


def _sc_edge_rows(src, dst, xl, xr, ee, att, lanes):
    n = xl.shape[0]
    e = ee.shape[0]
    CHA = 40
    epw = e // NW
    nch = epw // CHA
    zch = n // CHA

    mesh = plsc.VectorSubcoreMesh(core_axis_name="c", subcore_axis_name="s")

    BCH = 50
    nblk = nch // BCH

    buf = lambda: [
        pltpu.VMEM((CHA,), jnp.int32),
        pltpu.VMEM((CHA, 128), jnp.float32),
        pltpu.VMEM((CHA, 128), jnp.float32),
        pltpu.VMEM((CHA, 128), jnp.float32),
        pltpu.SemaphoreType.DMA,
        pltpu.SemaphoreType.DMA,
        pltpu.SemaphoreType.DMA,
    ]

    @functools.partial(
        pl.kernel,
        out_type=(
            jax.ShapeDtypeStruct((N_CORES, n, 128), jnp.float32),
            jax.ShapeDtypeStruct((e,), jnp.float32),
        ),
        mesh=mesh,
        scratch_types=buf() + buf() + [
            pltpu.VMEM((BCH * CHA,), jnp.int32),
            pltpu.VMEM((BCH * CHA,), jnp.int32),
            pltpu.VMEM((BCH * CHA,), jnp.float32),
            pltpu.VMEM((128,), jnp.float32),
            pltpu.VMEM((16,), jnp.int32),
            pltpu.VMEM((192,), jnp.float32),
            pltpu.VMEM_SHARED((n, 128), jnp.float32),
        ],
    )
    def k(src_h, dst_h, xl_h, xr_h, ee_h, att_h, lanes_h, orow_h, ex_h,
          *sc):
        bufA, bufB = sc[0:7], sc[7:14]
        src_blk, dst_blk, e_blk, att_v, lanes_v, s_v, acc_r = sc[14:21]
        c = lax.axis_index("c")
        s = lax.axis_index("s")
        wid = c * N_SUB + s

        pltpu.sync_copy(att_h, att_v)
        pltpu.sync_copy(lanes_h, lanes_v)
        att_regs = [att_v[pl.ds(r * 16, 16)] for r in range(8)]
        lanes = lanes_v[...]
        zero16 = jnp.zeros((16,), jnp.float32)
        for w in range(12):
            s_v[pl.ds(w * 16, 16)] = zero16

        z_v = bufA[1]

        @pl.loop(0, CHA)
        def _(i):
            for r in range(8):
                z_v[i, pl.ds(r * 16, 16)] = zero16

        @pl.loop(s, zch, step=N_SUB)
        def _(t):
            pltpu.sync_copy(z_v, acc_r.at[pl.ds(t * CHA, CHA), :])

        plsc.subcore_barrier()

        ebase0 = wid * epw

        def issue(rc, f, b):
            base = ebase0 + (f * BCH + 0) * CHA + rc * CHA
            siv = src_blk.at[pl.ds(rc * CHA, CHA)]
            div = dst_blk.at[pl.ds(rc * CHA, CHA)]
            pltpu.async_copy(xl_h.at[siv], b[1], b[4])
            pltpu.async_copy(xr_h.at[div], b[2], b[5])
            pltpu.async_copy(ee_h.at[pl.ds(base, CHA), :], b[3], b[6])

        def wait_gathers(b):
            siv = src_blk.at[pl.ds(0, CHA)]
            div = dst_blk.at[pl.ds(0, CHA)]
            pltpu.make_async_copy(xl_h.at[siv], b[1], b[4]).wait()
            pltpu.make_async_copy(xr_h.at[div], b[2], b[5]).wait()
            pltpu.make_async_copy(ee_h.at[pl.ds(0, CHA), :], b[3], b[6]).wait()

        def compute(rc, f, b):
            dstS, a_v, b_v, c_v = b[0:4]
            wait_gathers(b)
            base = ebase0 + (f * BCH) * CHA + rc * CHA
            off = rc * CHA
            dstS[pl.ds(0, 16)] = dst_blk[pl.ds(off, 16)]
            dstS[pl.ds(16, 16)] = dst_blk[pl.ds(off + 16, 16)]
            dstS[pl.ds(24, 16)] = dst_blk[pl.ds(off + 24, 16)]

            @pl.loop(0, CHA // 16)
            def _(g):
                def quad(qd, exg):
                    for u in range(4):
                        j = qd * 4 + u
                        i = g * 16 + j
                        w0 = 48 * u
                        acc = zero16
                        arows = []
                        for r in range(8):
                            av = a_v[i, pl.ds(r * 16, 16)]
                            m = (av + b_v[i, pl.ds(r * 16, 16)]
                                 + c_v[i, pl.ds(r * 16, 16)])
                            m = jnp.maximum(m, m * 0.2)
                            acc = acc + m * att_regs[r]
                            arows.append(av)
                        for kk in (8, 4, 2, 1):
                            s_v[pl.ds(w0 + 16, 16)] = acc
                            acc = (acc + s_v[pl.ds(w0 + 16 + kk, 16)]
                                   + s_v[pl.ds(w0 + kk, 16)])
                        ex = jnp.exp(acc)
                        for r in range(8):
                            a_v[i, pl.ds(r * 16, 16)] = arows[r] * ex
                        exg = jnp.where(lanes == j, ex, exg)
                    return exg

                exg = lax.fori_loop(0, 4, quad, zero16)
                e_blk[pl.ds(rc * CHA + g * 16, 16)] = exg

            pltpu.sync_copy(a_v, acc_r.at[dstS], add=True)

        for f in range(nblk):
            fb = ebase0 + f * BCH * CHA
            pltpu.sync_copy(src_h.at[pl.ds(fb, BCH * CHA)], src_blk)
            pltpu.sync_copy(dst_h.at[pl.ds(fb, BCH * CHA)], dst_blk)
            issue(0, f, bufA)

            @pl.loop(0, BCH, step=2)
            def _(rc):
                issue(rc + 1, f, bufB)
                compute(rc, f, bufA)
                issue(jnp.minimum(rc + 2, BCH - 1), f, bufA)
                compute(rc + 1, f, bufB)

            wait_gathers(bufA)
            pltpu.sync_copy(e_blk, ex_h.at[pl.ds(fb, BCH * CHA)])

        plsc.subcore_barrier()

        @pl.loop(s, zch, step=N_SUB)
        def _(t):
            pltpu.sync_copy(acc_r.at[pl.ds(t * CHA, CHA), :], z_v)
            pltpu.sync_copy(z_v, orow_h.at[c, pl.ds(t * CHA, CHA), :])

    return k(src, dst, xl, xr, ee, att, lanes)


def _sc_edge_denom(dst, ex, n, lanes):
    e = ex.shape[0]
    epw = e // NW
    nch = epw // CH
    zch = n // CH

    mesh = plsc.VectorSubcoreMesh(core_axis_name="c", subcore_axis_name="s")

    dbuf = lambda: [
        pltpu.VMEM((CH,), jnp.int32),
        pltpu.VMEM((CH,), jnp.float32),
        pltpu.VMEM((CH, 16), jnp.float32),
        pltpu.SemaphoreType.DMA,
        pltpu.SemaphoreType.DMA,
    ]

    @functools.partial(
        pl.kernel,
        out_type=jax.ShapeDtypeStruct((N_CORES, n, 16), jnp.float32),
        mesh=mesh,
        scratch_types=dbuf() + dbuf() + [
            pltpu.VMEM((16,), jnp.int32),
            pltpu.VMEM_SHARED((n, 16), jnp.float32),
        ],
    )
    def k(dst_h, ex_h, lanes_h, oden_h, *sc):
        bufA, bufB = sc[0:5], sc[5:10]
        lanes_v, acc_d = sc[10:12]
        c = lax.axis_index("c")
        s = lax.axis_index("s")
        wid = c * N_SUB + s

        pltpu.sync_copy(lanes_h, lanes_v)
        lanes = lanes_v[...]
        zero16 = jnp.zeros((16,), jnp.float32)
        zd_v = bufA[2]

        @pl.loop(0, CH)
        def _(i):
            zd_v[i, :] = zero16

        @pl.loop(s, zch, step=N_SUB)
        def _(t):
            pltpu.sync_copy(zd_v, acc_d.at[pl.ds(t * CH, CH), :])

        plsc.subcore_barrier()

        ebase0 = wid * epw

        def issue(ch, b):
            base = ebase0 + ch * CH
            pltpu.async_copy(dst_h.at[pl.ds(base, CH)], b[0], b[3])
            pltpu.async_copy(ex_h.at[pl.ds(base, CH)], b[1], b[4])

        def compute(ch, b):
            dst_v, e_v, d_v = b[0:3]
            pltpu.make_async_copy(dst_h.at[pl.ds(0, CH)], dst_v, b[3]).wait()
            pltpu.make_async_copy(ex_h.at[pl.ds(0, CH)], e_v, b[4]).wait()

            @pl.loop(0, CH // 16)
            def _(g):
                ex16 = e_v[pl.ds(g * 16, 16)]

                @pl.loop(0, 16)
                def _(j):
                    d_v[g * 16 + j, :] = jnp.where(lanes == j, ex16, 0.0)

            pltpu.sync_copy(d_v, acc_d.at[dst_v], add=True)

        issue(0, bufA)

        @pl.loop(0, nch - 1, step=2)
        def _(ch):
            issue(ch + 1, bufB)
            compute(ch, bufA)
            issue(ch + 2, bufA)
            compute(ch + 1, bufB)

        compute(nch - 1, bufA)

        plsc.subcore_barrier()

        @pl.loop(s, zch, step=N_SUB)
        def _(t):
            pltpu.sync_copy(acc_d.at[pl.ds(t * CH, CH), :], zd_v)
            pltpu.sync_copy(zd_v, oden_h.at[c, pl.ds(t * CH, CH), :])

    return k(dst, ex, lanes)


def kernel(x, edge_index, edge_attr, batch, Wl1, Wr1, We1, att1, b1,
           Wl2, Wr2, We2, att2, b2, Wfc, bfc):
    n = x.shape[0]
    nb = 64
    src = edge_index[0]
    dst = edge_index[1]
    lanes = jnp.arange(16, dtype=jnp.int32)

    xl1, xr1 = _tc_proj(x, Wl1, Wr1, blk=1000)
    e1, e2 = _tc_edge_emb(edge_attr, We1, We2, blk=8000)

    pr1, ex1 = _sc_edge_rows(src, dst, xl1, xr1, e1, att1, lanes)
    pd1 = _sc_edge_denom(dst, ex1, n, lanes)
    xl2, xr2 = _tc_combine_proj(pr1, pd1, b1, Wl2, Wr2, blk=1000)

    pr2, ex2 = _sc_edge_rows(src, dst, xl2, xr2, e2, att2, lanes)
    pd2 = _sc_edge_denom(dst, ex2, n, lanes)
    return _tc_final(pr2, pd2, b2, batch, Wfc, bfc, nb=nb, blk=1000)
